# baseline (speedup 1.0000x reference)
import functools

import jax
import jax.numpy as jnp
from jax import lax
from jax.experimental import pallas as pl
from jax.experimental.pallas import tpu as pltpu
from jax.experimental.pallas import tpu_sc as plsc

D = 1024
B = 4
S = 2048
EPS = 1e-5
NW = 32
NPH = 4
PPP = S // NPH
RPH = B * PPP
RPW = RPH // NW
CHG = 16
NCH = RPW // CHG
RPB = 512
BPB = PPP // RPB

_mesh = plsc.VectorSubcoreMesh(core_axis_name="c", subcore_axis_name="s")


@functools.partial(
    pl.kernel,
    mesh=_mesh,
    out_type=jax.ShapeDtypeStruct((RPH, D), jnp.float32),
    scratch_types=[
        pltpu.VMEM((RPW,), jnp.int32),
        pltpu.VMEM((RPW, D), jnp.float32),
        pltpu.SemaphoreType.DMA((NCH,)),
        pltpu.SemaphoreType.DMA((NCH,)),
    ],
)
def _sc_gather(ids_hbm, tok_hbm, out_hbm, idx_v, buf, gsem, ssem):
    wid = lax.axis_index("s") * 2 + lax.axis_index("c")
    r0 = wid * RPW

    pltpu.sync_copy(ids_hbm.at[pl.ds(r0, RPW)], idx_v)

    gathers = []
    for k in range(NCH):
        gathers.append(pltpu.async_copy(
            tok_hbm.at[idx_v.at[pl.ds(k * CHG, CHG)]],
            buf.at[pl.ds(k * CHG, CHG)], gsem.at[k]))

    stores = []
    for k in range(NCH):
        gathers[k].wait()
        stores.append(pltpu.async_copy(
            buf.at[pl.ds(k * CHG, CHG)],
            out_hbm.at[pl.ds(r0 + k * CHG, CHG)], ssem.at[k]))
    for k in range(NCH):
        stores[k].wait()


def _ln_math(t_ref, p_ref, g_ref, b_ref, o_ref):
    x = t_ref[...] + p_ref[...]
    mu = jnp.mean(x, axis=1, keepdims=True)
    xc = x - mu
    var = jnp.mean(xc * xc, axis=1, keepdims=True)
    o_ref[0] = xc * lax.rsqrt(var + EPS) * g_ref[...] + b_ref[...]


def _ln_first_body(t_ref, p_ref, g_ref, b_ref, o_ref):
    _ln_math(t_ref, p_ref, g_ref, b_ref, o_ref)


def _ln_next_body(acc_ref, t_ref, p_ref, g_ref, b_ref, o_ref):
    _ln_math(t_ref, p_ref, g_ref, b_ref, o_ref)


def _make_ln(q):
    data_specs = [
        pl.BlockSpec((RPB, D), lambda b, i: (b * BPB + i, 0)),
        pl.BlockSpec((RPB, D), lambda b, i: (q * BPB + i, 0)),
        pl.BlockSpec((1, D), lambda b, i: (0, 0)),
        pl.BlockSpec((1, D), lambda b, i: (0, 0)),
    ]
    out_spec = pl.BlockSpec((1, RPB, D), lambda b, i: (b, q * BPB + i, 0))
    out_shape = jax.ShapeDtypeStruct((B, S, D), jnp.float32)
    if q == 0:
        return pl.pallas_call(
            _ln_first_body, grid=(B, BPB), in_specs=data_specs,
            out_specs=out_spec, out_shape=out_shape)
    return pl.pallas_call(
        _ln_next_body, grid=(B, BPB),
        in_specs=[pl.BlockSpec(memory_space=pl.ANY)] + data_specs,
        out_specs=out_spec, out_shape=out_shape,
        input_output_aliases={0: 0})


_ln_calls = [_make_ln(q) for q in range(NPH)]


def kernel(input_ids, tok_table, pos_table, gamma, beta):
    ids = jnp.asarray(input_ids, jnp.int32)
    g2 = gamma.reshape(1, D)
    b2 = beta.reshape(1, D)
    out = None
    prev = _sc_gather(ids[:, 0:PPP].reshape(-1), tok_table)
    for q in range(NPH):
        if q + 1 < NPH:
            nxt = _sc_gather(
                ids[:, (q + 1) * PPP:(q + 2) * PPP].reshape(-1), tok_table)
        else:
            nxt = None
        if q == 0:
            out = _ln_calls[0](prev, pos_table, g2, b2)
        else:
            out = _ln_calls[q](out, prev, pos_table, g2, b2)
        prev = nxt
    return out


# speedup vs baseline: 1.0077x; 1.0077x over previous
import functools

import jax
import jax.numpy as jnp
from jax import lax
from jax.experimental import pallas as pl
from jax.experimental.pallas import tpu as pltpu
from jax.experimental.pallas import tpu_sc as plsc

D = 1024
B = 4
S = 2048
EPS = 1e-5
NW = 32
NPH = 4
PPP = S // NPH
RPH = B * PPP
RPW = RPH // NW
CHG = 32
RPB = 512
BPB = PPP // RPB

_mesh = plsc.VectorSubcoreMesh(core_axis_name="c", subcore_axis_name="s")


@functools.partial(
    pl.kernel,
    mesh=_mesh,
    out_type=jax.ShapeDtypeStruct((RPH, D), jnp.float32),
    scratch_types=[
        pltpu.VMEM((RPW,), jnp.int32),
        pltpu.VMEM((2 * CHG, D), jnp.float32),
        pltpu.SemaphoreType.DMA((2,)),
        pltpu.SemaphoreType.DMA((2,)),
    ],
)
def _sc_gather(ids_hbm, tok_hbm, out_hbm, idx_v, buf, gsem, ssem):
    wid = lax.axis_index("s") * 2 + lax.axis_index("c")
    r0 = wid * RPW

    pltpu.sync_copy(ids_hbm.at[pl.ds(r0, RPW)], idx_v)

    gathers = []
    for k in range(2):
        gathers.append(pltpu.async_copy(
            tok_hbm.at[idx_v.at[pl.ds(k * CHG, CHG)]],
            buf.at[pl.ds(k * CHG, CHG)], gsem.at[k]))

    stores = []
    for k in range(2):
        gathers[k].wait()
        stores.append(pltpu.async_copy(
            buf.at[pl.ds(k * CHG, CHG)],
            out_hbm.at[pl.ds(r0 + k * CHG, CHG)], ssem.at[k]))
    for k in range(2):
        stores[k].wait()


def _ln_math(t_ref, p_ref, g_ref, b_ref, o_ref):
    x = t_ref[...] + p_ref[...]
    mu = jnp.mean(x, axis=1, keepdims=True)
    xc = x - mu
    var = jnp.mean(xc * xc, axis=1, keepdims=True)
    o_ref[0] = xc * lax.rsqrt(var + EPS) * g_ref[...] + b_ref[...]


def _ln_first_body(t_ref, p_ref, g_ref, b_ref, o_ref):
    _ln_math(t_ref, p_ref, g_ref, b_ref, o_ref)


def _ln_next_body(acc_ref, t_ref, p_ref, g_ref, b_ref, o_ref):
    _ln_math(t_ref, p_ref, g_ref, b_ref, o_ref)


def _make_ln(q):
    data_specs = [
        pl.BlockSpec((RPB, D), lambda b, i: (b * BPB + i, 0)),
        pl.BlockSpec((RPB, D), lambda b, i: (q * BPB + i, 0)),
        pl.BlockSpec((1, D), lambda b, i: (0, 0)),
        pl.BlockSpec((1, D), lambda b, i: (0, 0)),
    ]
    out_spec = pl.BlockSpec((1, RPB, D), lambda b, i: (b, q * BPB + i, 0))
    out_shape = jax.ShapeDtypeStruct((B, S, D), jnp.float32)
    if q == 0:
        return pl.pallas_call(
            _ln_first_body, grid=(B, BPB), in_specs=data_specs,
            out_specs=out_spec, out_shape=out_shape)
    return pl.pallas_call(
        _ln_next_body, grid=(B, BPB),
        in_specs=[pl.BlockSpec(memory_space=pl.ANY)] + data_specs,
        out_specs=out_spec, out_shape=out_shape,
        input_output_aliases={0: 0})


_ln_calls = [_make_ln(q) for q in range(NPH)]


def kernel(input_ids, tok_table, pos_table, gamma, beta):
    ids = jnp.asarray(input_ids, jnp.int32)
    g2 = gamma.reshape(1, D)
    b2 = beta.reshape(1, D)
    out = None
    prev = _sc_gather(ids[:, 0:PPP].reshape(-1), tok_table)
    for q in range(NPH):
        if q + 1 < NPH:
            nxt = _sc_gather(
                ids[:, (q + 1) * PPP:(q + 2) * PPP].reshape(-1), tok_table)
        else:
            nxt = None
        if q == 0:
            out = _ln_calls[0](prev, pos_table, g2, b2)
        else:
            out = _ln_calls[q](out, prev, pos_table, g2, b2)
        prev = nxt
    return out
